# baseline (speedup 1.0000x reference)
import functools

import jax
import jax.numpy as jnp
import numpy as np
from jax import lax
from jax.experimental import pallas as pl
from jax.experimental.pallas import tpu as pltpu
from jax.experimental.pallas import tpu_sc as plsc

NBINS = 20
NCH = 16
NPAIR = (NCH * (NCH - 1)) // 2
JBINS = NBINS * NBINS
HBINS = NPAIR * JBINS
PAIRS = tuple((i, j) for i in range(NCH) for j in range(i + 1, NCH))
NSUB = 32
NLANES = 16
LW = 128
FOLD = LW // NCH
_HP = jax.lax.Precision.HIGHEST


def _mm_body(nblk, x_ref, mm_ref, mn_ref, mx_ref):
    k = pl.program_id(0)

    @pl.when(k == 0)
    def _init():
        mn_ref[...] = jnp.full(mn_ref.shape, jnp.inf, jnp.float32)
        mx_ref[...] = jnp.full(mx_ref.shape, -jnp.inf, jnp.float32)

    xb = x_ref[...]
    mn_ref[...] = jnp.minimum(mn_ref[...],
                              jnp.min(xb, axis=0, keepdims=True))
    mx_ref[...] = jnp.maximum(mx_ref[...],
                              jnp.max(xb, axis=0, keepdims=True))

    @pl.when(k == nblk - 1)
    def _fin():
        mm_ref[0:1, :] = mn_ref[...]
        mm_ref[1:2, :] = mx_ref[...]


def _make_hist_kernel(n):
    per = n // NSUB
    cs = min(4096, per)
    nchunks = per // cs
    groups = cs // NLANES
    mesh = plsc.VectorSubcoreMesh(core_axis_name="c", subcore_axis_name="s")

    @functools.partial(
        pl.kernel,
        mesh=mesh,
        compiler_params=pltpu.CompilerParams(needs_layout_passes=False),
        out_type=jax.ShapeDtypeStruct((NSUB, HBINS), jnp.float32),
        scratch_types=[
            pltpu.VMEM((NCH, cs), jnp.float32),
            pltpu.VMEM((2, NCH), jnp.float32),
            pltpu.VMEM((HBINS,), jnp.float32),
        ],
    )
    def hist_kernel(xt_hbm, mm_hbm, zeros_hbm, out_hbm, buf_v, mm_v, hist_v):
        cid = lax.axis_index("c")
        sid = lax.axis_index("s")
        wid = sid * 2 + cid
        base = wid * per
        pltpu.sync_copy(mm_hbm, mm_v)
        pltpu.sync_copy(zeros_hbm, hist_v)
        ones = jnp.full((NLANES,), 1.0, jnp.float32)
        lov = mm_v[0, pl.ds(0, NLANES)]
        hiv = mm_v[1, pl.ds(0, NLANES)]
        wv = (hiv - lov) / float(NBINS)
        los = [lov[c] for c in range(NCH)]
        ws = [wv[c] for c in range(NCH)]

        def chunk_body(t, carry):
            pltpu.sync_copy(xt_hbm.at[:, pl.ds(base + t * cs, cs)], buf_v)

            def group_body(g, c2):
                bv = []
                for c in range(NCH):
                    xv = buf_v[c, pl.ds(g * NLANES, NLANES)]
                    b = jnp.clip(((xv - los[c]) / ws[c]).astype(jnp.int32),
                                 0, NBINS - 1)
                    bv.append(b)
                b20 = [b * NBINS for b in bv]
                for p, (i, j) in enumerate(PAIRS):
                    idx = b20[i] + bv[j] + (JBINS * p)
                    plsc.addupdate_scatter(hist_v, [idx], ones)
                return c2

            lax.fori_loop(0, groups, group_body, 0)
            return carry

        lax.fori_loop(0, nchunks, chunk_body, 0)
        pltpu.sync_copy(hist_v, out_hbm.at[wid])

    return hist_kernel


def _final_body(ntot, part_ref, k1_ref, k1t_ref, k2_ref, k2t_ref,
                p1_ref, p2_ref, out_ref):
    J = jnp.sum(part_ref[...], axis=0)
    nf = float(ntot)
    mx = jnp.dot(J, k1_ref[...], preferred_element_type=jnp.float32,
                 precision=_HP)
    my = jnp.dot(J, k2_ref[...], preferred_element_type=jnp.float32,
                 precision=_HP)
    jp = J / nf
    pxe = jnp.dot(mx, k1t_ref[...], preferred_element_type=jnp.float32,
                  precision=_HP) / nf
    pye = jnp.dot(my, k2t_ref[...], preferred_element_type=jnp.float32,
                  precision=_HP) / nf
    M = jp * jnp.log(jp / (pxe * pye) + 1e-10)
    mi = jnp.sum(M, axis=1, keepdims=True)
    px = mx / nf
    py = my / nf
    entr = -jnp.sum(px * jnp.log(px + 1e-10), axis=1, keepdims=True)
    entc = -jnp.sum(py * jnp.log(py + 1e-10), axis=1, keepdims=True)
    vip = entr + entc - 2.0 * mi
    a1 = p1_ref[...] * vip
    a2 = p2_ref[...] * vip
    vi16 = (lax.dot_general(a1, p2_ref[...], (((0,), (0,)), ((), ())),
                            preferred_element_type=jnp.float32, precision=_HP)
            + lax.dot_general(a2, p1_ref[...], (((0,), (0,)), ((), ())),
                              preferred_element_type=jnp.float32,
                              precision=_HP))
    out_ref[...] = vi16


def _consts():
    c = np.arange(JBINS)
    k1 = (c[:, None] // NBINS == np.arange(NBINS)[None, :]).astype(np.float32)
    k2 = (c[:, None] % NBINS == np.arange(NBINS)[None, :]).astype(np.float32)
    p1 = np.zeros((NPAIR, NCH), np.float32)
    p2 = np.zeros((NPAIR, NCH), np.float32)
    for p, (i, j) in enumerate(PAIRS):
        p1[p, i] = 1.0
        p2[p, j] = 1.0
    return (jnp.asarray(k1), jnp.asarray(k1.T.copy()),
            jnp.asarray(k2), jnp.asarray(k2.T.copy()),
            jnp.asarray(p1), jnp.asarray(p2))


def kernel(inputs):
    B, T, A = inputs.shape
    n = B * T
    x = inputs.reshape(n, A)
    xt = x.T
    tblk = 8192 if n % 8192 == 0 else n
    nblk = n // tblk

    mm = pl.pallas_call(
        functools.partial(_mm_body, nblk),
        grid=(nblk,),
        in_specs=[pl.BlockSpec((tblk, NCH), lambda k: (k, 0))],
        out_specs=pl.BlockSpec((2, NCH), lambda k: (0, 0)),
        out_shape=jax.ShapeDtypeStruct((2, NCH), jnp.float32),
        scratch_shapes=[
            pltpu.VMEM((1, NCH), jnp.float32),
            pltpu.VMEM((1, NCH), jnp.float32),
        ],
        compiler_params=pltpu.CompilerParams(
            dimension_semantics=("arbitrary",)),
    )(x)

    partials = _make_hist_kernel(n)(
        xt, mm, jnp.zeros((HBINS,), jnp.float32))
    part3 = partials.reshape(NSUB, NPAIR, JBINS)

    k1, k1t, k2, k2t, p1, p2 = _consts()
    out16 = pl.pallas_call(
        functools.partial(_final_body, n),
        grid=(1,),
        in_specs=[
            pl.BlockSpec((NSUB, NPAIR, JBINS), lambda k: (0, 0, 0)),
            pl.BlockSpec((JBINS, NBINS), lambda k: (0, 0)),
            pl.BlockSpec((NBINS, JBINS), lambda k: (0, 0)),
            pl.BlockSpec((JBINS, NBINS), lambda k: (0, 0)),
            pl.BlockSpec((NBINS, JBINS), lambda k: (0, 0)),
            pl.BlockSpec((NPAIR, NCH), lambda k: (0, 0)),
            pl.BlockSpec((NPAIR, NCH), lambda k: (0, 0)),
        ],
        out_specs=pl.BlockSpec((NCH, NCH), lambda k: (0, 0)),
        out_shape=jax.ShapeDtypeStruct((NCH, NCH), jnp.float32),
    )(part3, k1, k1t, k2, k2t, p1, p2)
    return jnp.broadcast_to(out16[None, :, :], (B, A, A))


# speedup vs baseline: 1.1448x; 1.1448x over previous
import functools

import jax
import jax.numpy as jnp
import numpy as np
from jax import lax
from jax.experimental import pallas as pl
from jax.experimental.pallas import tpu as pltpu
from jax.experimental.pallas import tpu_sc as plsc

NBINS = 20
NCH = 16
NPAIR = (NCH * (NCH - 1)) // 2
JBINS = NBINS * NBINS
HBINS = NPAIR * JBINS
PAIRS = tuple((i, j) for i in range(NCH) for j in range(i + 1, NCH))
NSUB = 32
NLANES = 16
_T = [i * 15 - (i * (i - 1)) // 2 for i in range(NCH)]
_OFFI = [JBINS * (_T[i] - i - 1) for i in range(NCH)]
LW = 128
FOLD = LW // NCH
_HP = jax.lax.Precision.HIGHEST


def _prep_body(nblk, x_ref, xt_ref, mm_ref, mn_ref, mx_ref):
    k = pl.program_id(0)

    @pl.when(k == 0)
    def _init():
        mn_ref[...] = jnp.full(mn_ref.shape, jnp.inf, jnp.float32)
        mx_ref[...] = jnp.full(mx_ref.shape, -jnp.inf, jnp.float32)

    xb = x_ref[...]
    xt_ref[...] = xb.T
    mn_ref[...] = jnp.minimum(mn_ref[...],
                              jnp.min(xb, axis=0, keepdims=True))
    mx_ref[...] = jnp.maximum(mx_ref[...],
                              jnp.max(xb, axis=0, keepdims=True))

    @pl.when(k == nblk - 1)
    def _fin():
        mm_ref[0:1, :] = mn_ref[...]
        mm_ref[1:2, :] = mx_ref[...]


def _make_hist_kernel(n):
    per = n // NSUB
    cs = min(4096, per)
    nchunks = per // cs
    groups = cs // NLANES
    mesh = plsc.VectorSubcoreMesh(core_axis_name="c", subcore_axis_name="s")

    @functools.partial(
        pl.kernel,
        mesh=mesh,
        compiler_params=pltpu.CompilerParams(needs_layout_passes=False),
        out_type=jax.ShapeDtypeStruct((NSUB, HBINS), jnp.float32),
        scratch_types=[
            pltpu.VMEM((NCH, cs), jnp.float32),
            pltpu.VMEM((2, NCH), jnp.float32),
            pltpu.VMEM((HBINS,), jnp.float32),
        ],
    )
    def hist_kernel(xt_hbm, mm_hbm, zeros_hbm, out_hbm, buf_v, mm_v, hist_v):
        cid = lax.axis_index("c")
        sid = lax.axis_index("s")
        wid = sid * 2 + cid
        base = wid * per
        pltpu.sync_copy(mm_hbm, mm_v)
        pltpu.sync_copy(zeros_hbm, hist_v)
        ones = jnp.full((NLANES,), 1.0, jnp.float32)
        lov = mm_v[0, pl.ds(0, NLANES)]
        hiv = mm_v[1, pl.ds(0, NLANES)]
        wv = (hiv - lov) / float(NBINS)
        los = [lov[c] for c in range(NCH)]
        ws = [wv[c] for c in range(NCH)]

        def chunk_body(t, carry):
            pltpu.sync_copy(xt_hbm.at[:, pl.ds(base + t * cs, cs)], buf_v)

            def group_body(g, c2):
                bv = []
                for c in range(NCH):
                    xv = buf_v[c, pl.ds(g * NLANES, NLANES)]
                    b = jnp.clip(((xv - los[c]) / ws[c]).astype(jnp.int32),
                                 0, NBINS - 1)
                    bv.append(b)
                uu = [bv[i] * NBINS + _OFFI[i] for i in range(NCH)]
                vv = [bv[j] + JBINS * j for j in range(NCH)]
                for (i, j) in PAIRS:
                    plsc.addupdate_scatter(hist_v, [uu[i] + vv[j]], ones)
                return c2

            lax.fori_loop(0, groups, group_body, 0)
            return carry

        lax.fori_loop(0, nchunks, chunk_body, 0)
        pltpu.sync_copy(hist_v, out_hbm.at[wid])

    return hist_kernel


def _final_body(ntot, part_ref, k1_ref, k1t_ref, k2_ref, k2t_ref,
                p1_ref, p2_ref, out_ref):
    J = jnp.sum(part_ref[...], axis=0)
    nf = float(ntot)
    mx = jnp.dot(J, k1_ref[...], preferred_element_type=jnp.float32,
                 precision=_HP)
    my = jnp.dot(J, k2_ref[...], preferred_element_type=jnp.float32,
                 precision=_HP)
    jp = J / nf
    pxe = jnp.dot(mx, k1t_ref[...], preferred_element_type=jnp.float32,
                  precision=_HP) / nf
    pye = jnp.dot(my, k2t_ref[...], preferred_element_type=jnp.float32,
                  precision=_HP) / nf
    M = jp * jnp.log(jp / (pxe * pye) + 1e-10)
    mi = jnp.sum(M, axis=1, keepdims=True)
    px = mx / nf
    py = my / nf
    entr = -jnp.sum(px * jnp.log(px + 1e-10), axis=1, keepdims=True)
    entc = -jnp.sum(py * jnp.log(py + 1e-10), axis=1, keepdims=True)
    vip = entr + entc - 2.0 * mi
    a1 = p1_ref[...] * vip
    a2 = p2_ref[...] * vip
    vi16 = (lax.dot_general(a1, p2_ref[...], (((0,), (0,)), ((), ())),
                            preferred_element_type=jnp.float32, precision=_HP)
            + lax.dot_general(a2, p1_ref[...], (((0,), (0,)), ((), ())),
                              preferred_element_type=jnp.float32,
                              precision=_HP))
    out_ref[...] = vi16


def _consts():
    c = np.arange(JBINS)
    k1 = (c[:, None] // NBINS == np.arange(NBINS)[None, :]).astype(np.float32)
    k2 = (c[:, None] % NBINS == np.arange(NBINS)[None, :]).astype(np.float32)
    p1 = np.zeros((NPAIR, NCH), np.float32)
    p2 = np.zeros((NPAIR, NCH), np.float32)
    for p, (i, j) in enumerate(PAIRS):
        p1[p, i] = 1.0
        p2[p, j] = 1.0
    return (jnp.asarray(k1), jnp.asarray(k1.T.copy()),
            jnp.asarray(k2), jnp.asarray(k2.T.copy()),
            jnp.asarray(p1), jnp.asarray(p2))


def kernel(inputs):
    B, T, A = inputs.shape
    n = B * T
    x = inputs.reshape(n, A)
    tblk = 4096 if n % 4096 == 0 else n
    nblk = n // tblk

    xt, mm = pl.pallas_call(
        functools.partial(_prep_body, nblk),
        grid=(nblk,),
        in_specs=[pl.BlockSpec((tblk, NCH), lambda k: (k, 0))],
        out_specs=[
            pl.BlockSpec((NCH, tblk), lambda k: (0, k)),
            pl.BlockSpec((2, NCH), lambda k: (0, 0)),
        ],
        out_shape=[
            jax.ShapeDtypeStruct((NCH, n), jnp.float32),
            jax.ShapeDtypeStruct((2, NCH), jnp.float32),
        ],
        scratch_shapes=[
            pltpu.VMEM((1, NCH), jnp.float32),
            pltpu.VMEM((1, NCH), jnp.float32),
        ],
        compiler_params=pltpu.CompilerParams(
            dimension_semantics=("arbitrary",)),
    )(x)

    partials = _make_hist_kernel(n)(
        xt, mm, jnp.zeros((HBINS,), jnp.float32))
    part3 = partials.reshape(NSUB, NPAIR, JBINS)

    k1, k1t, k2, k2t, p1, p2 = _consts()
    out16 = pl.pallas_call(
        functools.partial(_final_body, n),
        grid=(1,),
        in_specs=[
            pl.BlockSpec((NSUB, NPAIR, JBINS), lambda k: (0, 0, 0)),
            pl.BlockSpec((JBINS, NBINS), lambda k: (0, 0)),
            pl.BlockSpec((NBINS, JBINS), lambda k: (0, 0)),
            pl.BlockSpec((JBINS, NBINS), lambda k: (0, 0)),
            pl.BlockSpec((NBINS, JBINS), lambda k: (0, 0)),
            pl.BlockSpec((NPAIR, NCH), lambda k: (0, 0)),
            pl.BlockSpec((NPAIR, NCH), lambda k: (0, 0)),
        ],
        out_specs=pl.BlockSpec((NCH, NCH), lambda k: (0, 0)),
        out_shape=jax.ShapeDtypeStruct((NCH, NCH), jnp.float32),
    )(part3, k1, k1t, k2, k2t, p1, p2)
    return jnp.broadcast_to(out16[None, :, :], (B, A, A))
